# Initial kernel scaffold; baseline (speedup 1.0000x reference)
import jax
import jax.numpy as jnp
from jax.experimental import pallas as pl


def kernel(tokens, embed_table, proj_w, norm_w):
    raise NotImplementedError("write your pallas kernel here")


# speedup vs baseline: 6.3009x; 6.3009x over previous
import functools

import jax
import jax.numpy as jnp
from jax import lax
from jax.experimental import pallas as pl
from jax.experimental.pallas import tpu as pltpu
from jax.experimental.pallas import tpu_sc as plsc

VOCAB = 32128
DIM = 896
BATCH = 4096
SEQ = 128
EPS = 1.1920928955078125e-07

LANES = 16
NUM_WORKERS = 32
BPW = BATCH // NUM_WORKERS
CHUNK = 64


def _proj_norm_body(nw_ref, emb_ref, w_ref, out_ref):
    x = emb_ref[...]
    y = lax.dot_general(x, w_ref[...], (((1,), (1,)), ((), ())),
                        preferred_element_type=jnp.float32)
    ms = jnp.mean(y * y, axis=1, keepdims=True)
    out_ref[...] = y * lax.rsqrt(ms + EPS) * nw_ref[...]


def _projected_table(embed_table, proj_w, norm_w):
    blk = 512
    return pl.pallas_call(
        _proj_norm_body,
        grid=(pl.cdiv(VOCAB, blk),),
        in_specs=[
            pl.BlockSpec((1, DIM), lambda i: (0, 0)),
            pl.BlockSpec((blk, DIM), lambda i: (i, 0)),
            pl.BlockSpec((DIM, DIM), lambda i: (0, 0)),
        ],
        out_specs=pl.BlockSpec((blk, DIM), lambda i: (i, 0)),
        out_shape=jax.ShapeDtypeStruct((VOCAB, DIM), jnp.float32),
    )(norm_w.reshape(1, DIM), embed_table, proj_w)


def _pool_body(tok_hbm, ntab_hbm, out_hbm, tok_v, rows_v, acc_v, sem):
    wid = lax.axis_index("s") * 2 + lax.axis_index("c")
    base = wid * BPW

    pltpu.sync_copy(tok_hbm.at[pl.ds(base, BPW)], tok_v)

    def row_body(i, _):
        for h in range(SEQ // CHUNK):
            idx = tok_v.at[i, pl.ds(h * CHUNK, CHUNK)]
            pltpu.async_copy(ntab_hbm.at[idx], rows_v, sem).wait()

            def chunk_body(c, _):
                sl = pl.ds(c * LANES, LANES)

                def add_row(j, acc):
                    return acc + rows_v[j, sl]

                acc = lax.fori_loop(0, CHUNK, add_row,
                                    jnp.zeros((LANES,), jnp.float32),
                                    unroll=8)
                if h == 0:
                    acc_v[sl] = acc
                else:
                    acc_v[sl] = acc_v[sl] + acc
                return 0

            lax.fori_loop(0, DIM // LANES, chunk_body, 0)

        def scale_body(c, _):
            sl = pl.ds(c * LANES, LANES)
            acc_v[sl] = acc_v[sl] * (1.0 / SEQ)
            return 0

        lax.fori_loop(0, DIM // LANES, scale_body, 0)
        pltpu.sync_copy(acc_v, out_hbm.at[base + i])
        return 0

    lax.fori_loop(0, BPW, row_body, 0)


def _pooled_lookup(tokens, ntab):
    mesh = plsc.VectorSubcoreMesh(core_axis_name="c", subcore_axis_name="s")
    run = functools.partial(
        pl.kernel, mesh=mesh,
        out_type=jax.ShapeDtypeStruct((BATCH, DIM), jnp.float32),
        scratch_types=[
            pltpu.VMEM((BPW, SEQ), jnp.int32),
            pltpu.VMEM((CHUNK, DIM), jnp.float32),
            pltpu.VMEM((DIM,), jnp.float32),
            pltpu.SemaphoreType.DMA,
        ],
    )(_pool_body)
    return run(tokens, ntab)


def kernel(tokens, embed_table, proj_w, norm_w):
    ntab = _projected_table(embed_table, proj_w, norm_w)
    return _pooled_lookup(tokens, ntab)


# speedup vs baseline: 9.9252x; 1.5752x over previous
import functools

import jax
import jax.numpy as jnp
from jax import lax
from jax.experimental import pallas as pl
from jax.experimental.pallas import tpu as pltpu
from jax.experimental.pallas import tpu_sc as plsc

VOCAB = 32128
DIM = 896
BATCH = 4096
SEQ = 128
EPS = 1.1920928955078125e-07

LANES = 16
NUM_WORKERS = 32
BPW = BATCH // NUM_WORKERS
CHUNK = 64


def _proj_norm_body(nw_ref, emb_ref, w_ref, out_ref):
    x = emb_ref[...]
    y = lax.dot_general(x, w_ref[...], (((1,), (1,)), ((), ())),
                        preferred_element_type=jnp.float32)
    ms = jnp.mean(y * y, axis=1, keepdims=True)
    out_ref[...] = y * lax.rsqrt(ms + EPS) * nw_ref[...]


def _projected_table(embed_table, proj_w, norm_w):
    blk = 512
    return pl.pallas_call(
        _proj_norm_body,
        grid=(pl.cdiv(VOCAB, blk),),
        in_specs=[
            pl.BlockSpec((1, DIM), lambda i: (0, 0)),
            pl.BlockSpec((blk, DIM), lambda i: (i, 0)),
            pl.BlockSpec((DIM, DIM), lambda i: (0, 0)),
        ],
        out_specs=pl.BlockSpec((blk, DIM), lambda i: (i, 0)),
        out_shape=jax.ShapeDtypeStruct((VOCAB, DIM), jnp.float32),
    )(norm_w.reshape(1, DIM), embed_table, proj_w)


HALF = BPW // 2


def _pool_body(tok_hbm, ntab_hbm, out_hbm,
               tok_v, buf0, buf1, acc_v, sem0, sem1, osem):
    wid = lax.axis_index("s") * 2 + lax.axis_index("c")
    base = wid * BPW

    def gstart(row, h, buf, sem):
        idx = tok_v.at[row, pl.ds(h * CHUNK, CHUNK)]
        pltpu.make_async_copy(ntab_hbm.at[idx], buf, sem).start()

    def gwait(buf, sem):
        idx = tok_v.at[0, pl.ds(0, CHUNK)]
        pltpu.make_async_copy(ntab_hbm.at[idx], buf, sem).wait()

    def accum(buf, slot, init):
        def chunk_body(c, _):
            sl = pl.ds(c * LANES, LANES)

            def add_row(j, a):
                return a + buf[j, sl]

            a = lax.fori_loop(0, CHUNK, add_row,
                              jnp.zeros((LANES,), jnp.float32), unroll=8)
            if init:
                acc_v[slot, sl] = a
            else:
                acc_v[slot, sl] = (acc_v[slot, sl] + a) * (1.0 / SEQ)
            return 0

        lax.fori_loop(0, DIM // LANES, chunk_body, 0)

    for p in range(BPW // HALF):
        prow = base + p * HALF
        pltpu.sync_copy(tok_hbm.at[pl.ds(prow, HALF)], tok_v)
        gstart(0, 0, buf0, sem0)
        gstart(0, 1, buf1, sem1)

        def row_body(g, _):
            slot = lax.rem(g, 2)

            @pl.when(g >= 2)
            def _():
                pltpu.make_async_copy(acc_v.at[slot], out_hbm.at[prow], osem).wait()

            gwait(buf0, sem0)
            accum(buf0, slot, init=True)

            @pl.when(g < HALF - 1)
            def _():
                gstart(g + 1, 0, buf0, sem0)

            gwait(buf1, sem1)
            accum(buf1, slot, init=False)

            @pl.when(g < HALF - 1)
            def _():
                gstart(g + 1, 1, buf1, sem1)

            pltpu.make_async_copy(acc_v.at[slot], out_hbm.at[prow + g], osem).start()
            return 0

        lax.fori_loop(0, HALF, row_body, 0)
        pltpu.make_async_copy(acc_v.at[0], out_hbm.at[prow], osem).wait()
        pltpu.make_async_copy(acc_v.at[0], out_hbm.at[prow], osem).wait()


def _pooled_lookup(tokens, ntab):
    mesh = plsc.VectorSubcoreMesh(core_axis_name="c", subcore_axis_name="s")
    run = functools.partial(
        pl.kernel, mesh=mesh,
        out_type=jax.ShapeDtypeStruct((BATCH, DIM), jnp.float32),
        scratch_types=[
            pltpu.VMEM((HALF, SEQ), jnp.int32),
            pltpu.VMEM((CHUNK, DIM), jnp.float32),
            pltpu.VMEM((CHUNK, DIM), jnp.float32),
            pltpu.VMEM((2, DIM), jnp.float32),
            pltpu.SemaphoreType.DMA,
            pltpu.SemaphoreType.DMA,
            pltpu.SemaphoreType.DMA,
        ],
    )(_pool_body)
    return run(tokens, ntab)


def kernel(tokens, embed_table, proj_w, norm_w):
    ntab = _projected_table(embed_table, proj_w, norm_w)
    return _pooled_lookup(tokens, ntab)
